# speedup vs baseline: 3.1421x; 3.1421x over previous
import functools

import jax
import jax.numpy as jnp
from jax.experimental import pallas as pl
from jax.experimental.pallas import tpu as pltpu

_N = 50000
_E = 800000
_BLK = 1000


def _pre_body(x_ref, feat_ref, dega_ref, degb_ref, w1x_ref, w1f_ref, b1_ref,
              g_ref, dinv_ref):
    deg = dega_ref[...] + degb_ref[...] + 1.0
    dinv = jax.lax.rsqrt(deg)
    h = (jnp.dot(x_ref[...], w1x_ref[...], precision=jax.lax.Precision.HIGHEST)
         + jnp.dot(feat_ref[...], w1f_ref[...], precision=jax.lax.Precision.HIGHEST)
         + b1_ref[...])
    g_ref[...] = dinv * h
    dinv_ref[...] = dinv


def _mid_body(acca_ref, accb_ref, g_ref, dinv_ref, wa_ref, wb_ref, b_ref,
              gout_ref):
    dinv = dinv_ref[...]
    pa = jnp.maximum(dinv * (acca_ref[...] + g_ref[:, :32]), 0.0)
    pb = jnp.maximum(dinv * (accb_ref[...] + g_ref[:, 32:]), 0.0)
    h = (jnp.dot(pa, wa_ref[...], precision=jax.lax.Precision.HIGHEST)
         + jnp.dot(pb, wb_ref[...], precision=jax.lax.Precision.HIGHEST)
         + b_ref[...])
    gout_ref[...] = dinv * h


def _fin_body(acca_ref, accb_ref, g_ref, dinv_ref, wa_ref, wb_ref, b_ref,
              out_ref):
    dinv = dinv_ref[...]
    pa = jnp.maximum(dinv * (acca_ref[...] + g_ref[:, :32]), 0.0)
    pb = jnp.maximum(dinv * (accb_ref[...] + g_ref[:, 32:]), 0.0)
    out_ref[...] = jnp.maximum(
        jnp.dot(pa, wa_ref[...], precision=jax.lax.Precision.HIGHEST)
        + jnp.dot(pb, wb_ref[...], precision=jax.lax.Precision.HIGHEST)
        + b_ref[...], 0.0)


def _row_spec(w):
    return pl.BlockSpec((_BLK, w), lambda i: (i, 0))


def _full_spec(shape):
    return pl.BlockSpec(shape, lambda i: tuple(0 for _ in shape))


_GRID = _N // _BLK


def _pre_call(x, feat, dega, degb, w1x, w1f, b1):
    return pl.pallas_call(
        _pre_body,
        grid=(_GRID,),
        in_specs=[_row_spec(64), _row_spec(64), _row_spec(1), _row_spec(1),
                  _full_spec((64, 64)), _full_spec((64, 64)),
                  _full_spec((1, 64))],
        out_specs=[_row_spec(64), _row_spec(1)],
        out_shape=[jax.ShapeDtypeStruct((_N, 64), jnp.float32),
                   jax.ShapeDtypeStruct((_N, 1), jnp.float32)],
    )(x, feat, dega, degb, w1x, w1f, b1)


def _mid_call(acca, accb, g, dinv, wa, wb, b):
    return pl.pallas_call(
        _mid_body,
        grid=(_GRID,),
        in_specs=[_row_spec(32), _row_spec(32), _row_spec(64), _row_spec(1),
                  _full_spec((32, 64)), _full_spec((32, 64)),
                  _full_spec((1, 64))],
        out_specs=_row_spec(64),
        out_shape=jax.ShapeDtypeStruct((_N, 64), jnp.float32),
    )(acca, accb, g, dinv, wa, wb, b)


def _fin_call(acca, accb, g, dinv, wa, wb, b):
    return pl.pallas_call(
        _fin_body,
        grid=(_GRID,),
        in_specs=[_row_spec(32), _row_spec(32), _row_spec(64), _row_spec(1),
                  _full_spec((32, 64)), _full_spec((32, 64)),
                  _full_spec((1, 64))],
        out_specs=_row_spec(64),
        out_shape=jax.ShapeDtypeStruct((_N, 64), jnp.float32),
    )(acca, accb, g, dinv, wa, wb, b)


def _edge_pass(g, row, col):
    acc = jnp.zeros((_N, 64), jnp.float32).at[col].add(g[row])
    return acc[:, :32], acc[:, 32:]


def kernel(x, feat, edge_index, W1, b1, W2, b2, Wfc, bfc):
    row = edge_index[0]
    col = edge_index[1]

    deg = jnp.zeros((_N, 1), jnp.float32).at[row, 0].add(1.0)
    dega = deg
    degb = jnp.zeros((_N, 1), jnp.float32)

    w1x = W1[:, :64].T
    w1f = W1[:, 64:].T
    g1, dinv = _pre_call(x, feat, dega, degb, w1x, w1f, b1.reshape(1, 64))

    acc1a, acc1b = _edge_pass(g1, row, col)
    g2 = _mid_call(acc1a, acc1b, g1, dinv, W2[:, :32].T, W2[:, 32:].T,
                   b2.reshape(1, 64))

    acc2a, acc2b = _edge_pass(g2, row, col)
    out = _fin_call(acc2a, acc2b, g2, dinv, Wfc[:, :32].T, Wfc[:, 32:].T,
                    bfc.reshape(1, 64))
    return out


# speedup vs baseline: 11.9393x; 3.7998x over previous
import functools

import jax
import jax.numpy as jnp
from jax import lax
from jax.experimental import pallas as pl
from jax.experimental.pallas import tpu as pltpu
from jax.experimental.pallas import tpu_sc as plsc

_N = 50000
_E = 800000
_BLK = 1000
_GRID = _N // _BLK

_NCH = _E // 128
_T = 400
_PCHT = 16 * _T
_PAD = _PCHT - _NCH
_G = 8
_NGRP = _T // _G
_NACC = 64000
_TROW = _NACC // 16
_RCH = 800

_DT = 208
_DPCH = 32 * _DT
_DPAD = _DPCH - _NCH
_DG = 16
_DNG = _DT // _DG

_mesh = plsc.VectorSubcoreMesh(core_axis_name="c", subcore_axis_name="s")
_sc_params = pltpu.CompilerParams(use_tc_tiling_on_sc=False)


@functools.partial(
    pl.kernel,
    out_type=jax.ShapeDtypeStruct((2 * _NACC, 8), jnp.float32),
    mesh=_mesh,
    compiler_params=_sc_params,
    scratch_types=[
        pltpu.VMEM((_DG, 128), jnp.int32),
        pltpu.VMEM((128, 8), jnp.float32),
        pltpu.VMEM((_RCH, 8), jnp.float32),
        pltpu.VMEM_SHARED((_NACC, 8), jnp.float32),
    ],
)
def _deg_kernel(row_hbm, ones_hbm, zeros_hbm, out_hbm, idx_v, ones_v, zb_v,
                deg_sh):
    c = lax.axis_index("c")
    s = lax.axis_index("s")
    pltpu.sync_copy(zeros_hbm, zb_v)
    for k in range(5):
        pltpu.sync_copy(zb_v, deg_sh.at[pl.ds(s * _TROW + k * _RCH, _RCH)])
    pltpu.sync_copy(ones_hbm, ones_v)
    plsc.subcore_barrier()

    base = (s * 2 + c) * _DT

    def body(m, carry):
        pltpu.sync_copy(row_hbm.at[pl.ds(base + m * _DG, _DG)], idx_v)
        for j in range(_DG):
            pltpu.sync_copy(ones_v, deg_sh.at[idx_v.at[j]], add=True)
        return carry

    lax.fori_loop(0, _DNG, body, 0)
    plsc.subcore_barrier()
    for k in range(5):
        pltpu.sync_copy(deg_sh.at[pl.ds(s * _TROW + k * _RCH, _RCH)], zb_v)
        pltpu.sync_copy(zb_v,
                        out_hbm.at[pl.ds(c * _NACC + s * _TROW + k * _RCH,
                                         _RCH)])


@functools.partial(
    pl.kernel,
    out_type=jax.ShapeDtypeStruct((4 * _NACC, 16), jnp.float32),
    mesh=_mesh,
    compiler_params=_sc_params,
    scratch_types=[
        pltpu.VMEM((2 * _G, 128), jnp.int32),
        pltpu.VMEM((2 * _G, 128), jnp.int32),
        pltpu.VMEM((2 * _G, 128, 16), jnp.float32),
        pltpu.VMEM((_RCH, 16), jnp.float32),
        pltpu.VMEM_SHARED((_NACC, 16), jnp.float32),
        pltpu.SemaphoreType.DMA,
    ],
)
def _edge_kernel(gidx_hbm, col_hbm, table_hbm, zeros_hbm, out_hbm,
                 gi_v, col_v, rows_v, zb_v, acc_sh, sem):
    c = lax.axis_index("c")
    s = lax.axis_index("s")

    for q in range(2):
        qid = c * 2 + q

        pltpu.sync_copy(zeros_hbm, zb_v)
        for k in range(5):
            pltpu.sync_copy(zb_v, acc_sh.at[pl.ds(s * _TROW + k * _RCH,
                                                  _RCH)])
        plsc.subcore_barrier()

        ibase = qid * _PCHT + s * _T
        cbase = s * _T

        def stage_fire(m, boff):
            pltpu.sync_copy(gidx_hbm.at[pl.ds(ibase + m * _G, _G)],
                            gi_v.at[pl.ds(boff, _G)])
            pltpu.sync_copy(col_hbm.at[pl.ds(cbase + m * _G, _G)],
                            col_v.at[pl.ds(boff, _G)])
            for j in range(_G):
                pltpu.async_copy(table_hbm.at[gi_v.at[boff + j]],
                                 rows_v.at[boff + j], sem)

        def drain(boff):
            for j in range(_G):
                pltpu.make_async_copy(table_hbm.at[gi_v.at[boff + j]],
                                      rows_v.at[boff + j], sem).wait()

        def scatter(boff):
            for j in range(_G):
                pltpu.sync_copy(rows_v.at[boff + j],
                                acc_sh.at[col_v.at[boff + j]], add=True)

        stage_fire(0, 0)

        def body(m, carry):
            b0 = pl.multiple_of(lax.rem(m, 2) * _G, _G)
            b1 = pl.multiple_of(_G - b0, _G)
            drain(b0)
            stage_fire(m + 1, b1)
            scatter(b0)
            return carry

        lax.fori_loop(0, _NGRP - 1, body, 0)
        last = ((_NGRP - 1) % 2) * _G
        drain(last)
        scatter(last)

        plsc.subcore_barrier()
        for k in range(5):
            pltpu.sync_copy(acc_sh.at[pl.ds(s * _TROW + k * _RCH, _RCH)],
                            zb_v)
            pltpu.sync_copy(
                zb_v,
                out_hbm.at[pl.ds(qid * _NACC + s * _TROW + k * _RCH, _RCH)])
        plsc.subcore_barrier()


def _pre_body(x_ref, feat_ref, dega_ref, degb_ref, w1x_ref, w1f_ref, b1_ref,
              g_ref, dinv_ref):
    deg = dega_ref[:, :1] + degb_ref[:, :1] + 1.0
    dinv = jax.lax.rsqrt(deg)
    h = (jnp.dot(x_ref[...], w1x_ref[...], precision=jax.lax.Precision.HIGHEST)
         + jnp.dot(feat_ref[...], w1f_ref[...], precision=jax.lax.Precision.HIGHEST)
         + b1_ref[...])
    g_ref[...] = dinv * h
    dinv_ref[...] = dinv


def _acc_combine(acc_refs, g_ref, dinv, w_ref):
    h = None
    for q, aq in enumerate(acc_refs):
        pq = jnp.maximum(dinv * (aq[...] + g_ref[:, 16 * q:16 * q + 16]), 0.0)
        t = jnp.dot(pq, w_ref[16 * q:16 * q + 16, :],
                    precision=jax.lax.Precision.HIGHEST)
        h = t if h is None else h + t
    return h


def _mid_body(a0, a1, a2, a3, g_ref, dinv_ref, w_ref, b_ref, gout_ref):
    dinv = dinv_ref[...]
    h = _acc_combine((a0, a1, a2, a3), g_ref, dinv, w_ref) + b_ref[...]
    gout_ref[...] = dinv * h


def _fin_body(a0, a1, a2, a3, g_ref, dinv_ref, w_ref, b_ref, out_ref):
    dinv = dinv_ref[...]
    h = _acc_combine((a0, a1, a2, a3), g_ref, dinv, w_ref) + b_ref[...]
    out_ref[...] = jnp.maximum(h, 0.0)


def _row_spec(w):
    return pl.BlockSpec((_BLK, w), lambda i: (i, 0))


def _part_spec(w, part):
    off = part * (_NACC // _BLK)
    return pl.BlockSpec((_BLK, w), lambda i, _o=off: (i + _o, 0))


def _full_spec(shape):
    return pl.BlockSpec(shape, lambda i: tuple(0 for _ in shape))


def _pre_call(x, feat, degp, w1x, w1f, b1):
    return pl.pallas_call(
        _pre_body,
        grid=(_GRID,),
        in_specs=[_row_spec(64), _row_spec(64), _part_spec(8, 0),
                  _part_spec(8, 1), _full_spec((64, 64)),
                  _full_spec((64, 64)), _full_spec((1, 64))],
        out_specs=[_row_spec(64), _row_spec(1)],
        out_shape=[jax.ShapeDtypeStruct((_N, 64), jnp.float32),
                   jax.ShapeDtypeStruct((_N, 1), jnp.float32)],
    )(x, feat, degp, degp, w1x, w1f, b1)


def _combine_call(body, acc, g, dinv, wt, b):
    return pl.pallas_call(
        body,
        grid=(_GRID,),
        in_specs=[_part_spec(16, 0), _part_spec(16, 1), _part_spec(16, 2),
                  _part_spec(16, 3), _row_spec(64), _row_spec(1),
                  _full_spec((64, 64)), _full_spec((1, 64))],
        out_specs=_row_spec(64),
        out_shape=jax.ShapeDtypeStruct((_N, 64), jnp.float32),
    )(acc, acc, acc, acc, g, dinv, wt, b)


def kernel(x, feat, edge_index, W1, b1, W2, b2, Wfc, bfc):
    row = edge_index[0]
    col = edge_index[1]
    row2d = jnp.pad(row.reshape(_NCH, 128), ((0, _DPAD), (0, 0)),
                    constant_values=_N)
    col2d = jnp.pad(col.reshape(_NCH, 128), ((0, _PAD), (0, 0)),
                    constant_values=_N)
    pad0 = jnp.zeros((_PAD, 128), jnp.int32)
    r4 = row * 4
    gidx = jnp.concatenate(
        [jnp.concatenate([(r4 + q).reshape(_NCH, 128), pad0], axis=0)
         for q in range(4)], axis=0)
    ones8 = jnp.ones((128, 8), jnp.float32)
    zeros8 = jnp.zeros((_RCH, 8), jnp.float32)
    zeros16 = jnp.zeros((_RCH, 16), jnp.float32)

    degp = _deg_kernel(row2d, ones8, zeros8)

    g1, dinv = _pre_call(x, feat, degp, W1[:, :64].T, W1[:, 64:].T,
                         b1.reshape(1, 64))
    acc1 = _edge_kernel(gidx, col2d, g1.reshape(4 * _N, 16), zeros16)
    g2 = _combine_call(_mid_body, acc1, g1, dinv, W2.T, b2.reshape(1, 64))
    acc2 = _edge_kernel(gidx, col2d, g2.reshape(4 * _N, 16), zeros16)
    out = _combine_call(_fin_body, acc2, g2, dinv, Wfc.T, bfc.reshape(1, 64))
    return out


# speedup vs baseline: 15.7933x; 1.3228x over previous
import functools

import jax
import jax.numpy as jnp
from jax import lax
from jax.experimental import pallas as pl
from jax.experimental.pallas import tpu as pltpu
from jax.experimental.pallas import tpu_sc as plsc

_N = 50000
_E = 800000
_BLK = 784
_GRID = 64

_NCH = _E // 128
_T = 400
_PCHT = 16 * _T
_PAD = _PCHT - _NCH
_NB = _T // 8
_R = 4
_NACC = 50176
_TROW = _NACC // 16
_RCH = 112
_DRCH = 448

_DT = 208
_DPCH = 32 * _DT
_DPAD = _DPCH - _NCH
_DG = 16
_DNG = _DT // _DG

_mesh = plsc.VectorSubcoreMesh(core_axis_name="c", subcore_axis_name="s")
_sc_params = pltpu.CompilerParams(use_tc_tiling_on_sc=False)


@functools.partial(
    pl.kernel,
    out_type=jax.ShapeDtypeStruct((2 * _NACC, 8), jnp.float32),
    mesh=_mesh,
    compiler_params=_sc_params,
    scratch_types=[
        pltpu.VMEM((_DG, 128), jnp.int32),
        pltpu.VMEM((128, 8), jnp.float32),
        pltpu.VMEM((_DRCH, 8), jnp.float32),
        pltpu.VMEM_SHARED((_NACC, 8), jnp.float32),
    ],
)
def _deg_kernel(row_hbm, ones_hbm, zeros_hbm, out_hbm, idx_v, ones_v, zb_v,
                deg_sh):
    c = lax.axis_index("c")
    s = lax.axis_index("s")
    pltpu.sync_copy(zeros_hbm, zb_v)
    for k in range(7):
        pltpu.sync_copy(zb_v, deg_sh.at[pl.ds(s * _TROW + k * _DRCH, _DRCH)])
    pltpu.sync_copy(ones_hbm, ones_v)
    plsc.subcore_barrier()

    base = (s * 2 + c) * _DT

    def body(m, carry):
        pltpu.sync_copy(row_hbm.at[pl.ds(base + m * _DG, _DG)], idx_v)
        for j in range(_DG):
            pltpu.sync_copy(ones_v, deg_sh.at[idx_v.at[j]], add=True)
        return carry

    lax.fori_loop(0, _DNG, body, 0)
    plsc.subcore_barrier()
    for k in range(7):
        pltpu.sync_copy(deg_sh.at[pl.ds(s * _TROW + k * _DRCH, _DRCH)], zb_v)
        pltpu.sync_copy(zb_v,
                        out_hbm.at[pl.ds(c * _NACC + s * _TROW + k * _DRCH,
                                         _DRCH)])


@functools.partial(
    pl.kernel,
    out_type=jax.ShapeDtypeStruct((2 * _NACC, 32), jnp.float32),
    mesh=_mesh,
    compiler_params=_sc_params,
    scratch_types=[
        pltpu.VMEM((16, 128), jnp.int32),
        pltpu.VMEM((16, 128), jnp.int32),
        pltpu.VMEM((_R, 128, 32), jnp.float32),
        pltpu.VMEM((_RCH, 32), jnp.float32),
        pltpu.VMEM_SHARED((_NACC, 32), jnp.float32),
        pltpu.SemaphoreType.DMA,
    ],
)
def _edge_kernel(gidx_hbm, col_hbm, table_hbm, zeros_hbm, out_hbm,
                 gi_v, col_v, rows_v, zb_v, acc_sh, sem):
    c = lax.axis_index("c")
    s = lax.axis_index("s")

    pltpu.sync_copy(zeros_hbm, zb_v)
    for k in range(28):
        pltpu.sync_copy(zb_v, acc_sh.at[pl.ds(s * _TROW + k * _RCH, _RCH)])
    plsc.subcore_barrier()

    ibase = c * _PCHT + s * _T
    cbase = s * _T

    def stage(b, half):
        pltpu.sync_copy(gidx_hbm.at[pl.ds(ibase + b * 8, 8)],
                        gi_v.at[pl.ds(half * 8, 8)])
        pltpu.sync_copy(col_hbm.at[pl.ds(cbase + b * 8, 8)],
                        col_v.at[pl.ds(half * 8, 8)])

    def fire(k):
        pltpu.async_copy(table_hbm.at[gi_v.at[lax.rem(k, 16)]],
                         rows_v.at[lax.rem(k, _R)], sem)

    def drain(k):
        pltpu.make_async_copy(table_hbm.at[gi_v.at[lax.rem(k, 16)]],
                              rows_v.at[lax.rem(k, _R)], sem).wait()

    def scatter(k):
        pltpu.sync_copy(rows_v.at[lax.rem(k, _R)],
                        acc_sh.at[col_v.at[lax.rem(k, 16)]], add=True)

    stage(0, 0)
    for k0 in range(_R):
        fire(k0)

    def body(k, carry):
        b_next = k // 8 + 1

        @pl.when((lax.rem(k, 8) == 0) & (b_next < _NB))
        def _():
            stage(b_next, lax.rem(b_next, 2))

        drain(k)
        scatter(k)

        @pl.when(k + _R < _T)
        def _():
            fire(k + _R)

        return carry

    lax.fori_loop(0, _T, body, 0)
    plsc.subcore_barrier()
    for k in range(28):
        pltpu.sync_copy(acc_sh.at[pl.ds(s * _TROW + k * _RCH, _RCH)], zb_v)
        pltpu.sync_copy(
            zb_v,
            out_hbm.at[pl.ds(c * _NACC + s * _TROW + k * _RCH, _RCH)])


def _pre_body(x_ref, feat_ref, dega_ref, degb_ref, w1x_ref, w1f_ref, b1_ref,
              g_ref, dinv_ref):
    deg = dega_ref[:, :1] + degb_ref[:, :1] + 1.0
    dinv = jax.lax.rsqrt(deg)
    h = (jnp.dot(x_ref[...], w1x_ref[...], precision=jax.lax.Precision.HIGHEST)
         + jnp.dot(feat_ref[...], w1f_ref[...], precision=jax.lax.Precision.HIGHEST)
         + b1_ref[...])
    g_ref[...] = dinv * h
    dinv_ref[...] = dinv


def _mid_body(acca_ref, accb_ref, g_ref, dinv_ref, wa_ref, wb_ref, b_ref,
              gout_ref):
    dinv = dinv_ref[...]
    pa = jnp.maximum(dinv * (acca_ref[...] + g_ref[:, :32]), 0.0)
    pb = jnp.maximum(dinv * (accb_ref[...] + g_ref[:, 32:]), 0.0)
    h = (jnp.dot(pa, wa_ref[...], precision=jax.lax.Precision.HIGHEST)
         + jnp.dot(pb, wb_ref[...], precision=jax.lax.Precision.HIGHEST)
         + b_ref[...])
    gout_ref[...] = dinv * h


def _fin_body(acca_ref, accb_ref, g_ref, dinv_ref, wa_ref, wb_ref, b_ref,
              out_ref):
    dinv = dinv_ref[...]
    pa = jnp.maximum(dinv * (acca_ref[...] + g_ref[:, :32]), 0.0)
    pb = jnp.maximum(dinv * (accb_ref[...] + g_ref[:, 32:]), 0.0)
    out_ref[...] = jnp.maximum(
        jnp.dot(pa, wa_ref[...], precision=jax.lax.Precision.HIGHEST)
        + jnp.dot(pb, wb_ref[...], precision=jax.lax.Precision.HIGHEST)
        + b_ref[...], 0.0)


def _row_spec(w):
    return pl.BlockSpec((_BLK, w), lambda i: (i, 0))


def _part_spec(w, part):
    off = part * (_NACC // _BLK)
    return pl.BlockSpec((_BLK, w), lambda i, _o=off: (i + _o, 0))


def _full_spec(shape):
    return pl.BlockSpec(shape, lambda i: tuple(0 for _ in shape))


def _pre_call(x, feat, degp, w1x, w1f, b1):
    return pl.pallas_call(
        _pre_body,
        grid=(_GRID,),
        in_specs=[_row_spec(64), _row_spec(64), _part_spec(8, 0),
                  _part_spec(8, 1), _full_spec((64, 64)),
                  _full_spec((64, 64)), _full_spec((1, 64))],
        out_specs=[_row_spec(64), _row_spec(1)],
        out_shape=[jax.ShapeDtypeStruct((_N, 64), jnp.float32),
                   jax.ShapeDtypeStruct((_N, 1), jnp.float32)],
    )(x, feat, degp, degp, w1x, w1f, b1)


def _combine_call(body, acc, g, dinv, wa, wb, b):
    return pl.pallas_call(
        body,
        grid=(_GRID,),
        in_specs=[_part_spec(32, 0), _part_spec(32, 1), _row_spec(64),
                  _row_spec(1), _full_spec((32, 64)), _full_spec((32, 64)),
                  _full_spec((1, 64))],
        out_specs=_row_spec(64),
        out_shape=jax.ShapeDtypeStruct((_N, 64), jnp.float32),
    )(acc, acc, g, dinv, wa, wb, b)


def kernel(x, feat, edge_index, W1, b1, W2, b2, Wfc, bfc):
    row = edge_index[0]
    col = edge_index[1]
    row2d = jnp.pad(row.reshape(_NCH, 128), ((0, _DPAD), (0, 0)),
                    constant_values=_N)
    col2d = jnp.pad(col.reshape(_NCH, 128), ((0, _PAD), (0, 0)),
                    constant_values=_N)
    pad0 = jnp.zeros((_PAD, 128), jnp.int32)
    gidx = jnp.concatenate([(row * 2).reshape(_NCH, 128), pad0,
                            (row * 2 + 1).reshape(_NCH, 128), pad0], axis=0)
    ones8 = jnp.ones((128, 8), jnp.float32)
    zeros8 = jnp.zeros((_DRCH, 8), jnp.float32)
    zeros32 = jnp.zeros((_RCH, 32), jnp.float32)

    degp = _deg_kernel(row2d, ones8, zeros8)

    g1, dinv = _pre_call(x, feat, degp, W1[:, :64].T, W1[:, 64:].T,
                         b1.reshape(1, 64))
    acc1 = _edge_kernel(gidx, col2d, g1.reshape(2 * _N, 32), zeros32)
    g2 = _combine_call(_mid_body, acc1, g1, dinv, W2[:, :32].T,
                       W2[:, 32:].T, b2.reshape(1, 64))
    acc2 = _edge_kernel(gidx, col2d, g2.reshape(2 * _N, 32), zeros32)
    out = _combine_call(_fin_body, acc2, g2, dinv, Wfc[:, :32].T,
                        Wfc[:, 32:].T, bfc.reshape(1, 64))
    return out
